# baseline (speedup 1.0000x reference)
import jax
import jax.numpy as jnp
from jax.experimental import pallas as pl
from jax.experimental.pallas import tpu as pltpu

_BLK = 1152


def _vq_kernel(x_ref, p_ref, cb_ref, out_ref, cbn_ref, s_ref):
    i = pl.program_id(0)

    @pl.when(i == 0)
    def _():
        cb = cb_ref[...]
        cn = jnp.sqrt(jnp.sum(cb * cb, axis=1, keepdims=True))
        cbn_ref[...] = cb / (cn + 1e-12)

    x = x_ref[...]
    p = p_ref[...]
    proj = jax.lax.dot_general(
        x, p, (((1,), (0,)), ((), ())), preferred_element_type=jnp.float32)
    qn = jnp.sqrt(jnp.sum(proj * proj, axis=1, keepdims=True))
    qx = proj / (qn + 1e-12)
    scores = jax.lax.dot_general(
        qx, cbn_ref[...], (((1,), (1,)), ((), ())),
        preferred_element_type=jnp.float32)

    prev = s_ref[(i + 1) % 2]
    idx = jnp.argmax(prev, axis=1)
    out_ref[...] = idx.reshape(_BLK, 1).astype(jnp.int32)

    s_ref[i % 2] = scores


def kernel(x, rand_projs, codebook):
    b, n, dim = x.shape
    h, k, e = codebook.shape
    ntok = b * n
    xf = x.reshape(ntok, dim)
    p = rand_projs.reshape(dim, e)
    cb = codebook.reshape(k, e)
    nb = ntok // _BLK
    out = pl.pallas_call(
        _vq_kernel,
        grid=(nb + 1,),
        in_specs=[
            pl.BlockSpec((_BLK, dim), lambda i: (jnp.minimum(i, nb - 1), 0)),
            pl.BlockSpec((dim, e), lambda i: (0, 0)),
            pl.BlockSpec((k, e), lambda i: (0, 0)),
        ],
        out_specs=pl.BlockSpec((_BLK, 1), lambda i: (jnp.maximum(i - 1, 0), 0)),
        out_shape=jax.ShapeDtypeStruct((ntok, 1), jnp.int32),
        scratch_shapes=[
            pltpu.VMEM((k, e), jnp.float32),
            pltpu.VMEM((2, _BLK, k), jnp.float32),
        ],
        compiler_params=pltpu.CompilerParams(
            dimension_semantics=("arbitrary",)),
    )(xf, p, cb)
    return out.reshape(b, n)


# speedup vs baseline: 1.1360x; 1.1360x over previous
import jax
import jax.numpy as jnp
from jax.experimental import pallas as pl
from jax.experimental.pallas import tpu as pltpu

_BLK = 1152


def _vq_kernel(x_ref, p_ref, cb_ref, out_ref, cbn_ref):
    i = pl.program_id(0)

    @pl.when(i == 0)
    def _():
        cb = cb_ref[...]
        cn = jnp.sqrt(jnp.sum(cb * cb, axis=1, keepdims=True))
        cbn_ref[...] = cb / (cn + 1e-12)

    x = x_ref[...]
    p = p_ref[...]
    proj = jax.lax.dot_general(
        x, p, (((1,), (0,)), ((), ())), preferred_element_type=jnp.float32)
    qn = jnp.sqrt(jnp.sum(proj * proj, axis=1, keepdims=True))
    qx = proj / (qn + 1e-12)
    scores_t = jax.lax.dot_general(
        cbn_ref[...], qx, (((1,), (1,)), ((), ())),
        preferred_element_type=jnp.float32)
    idx = jnp.argmax(scores_t, axis=0)
    out_ref[...] = idx.reshape(1, 1, -1).astype(jnp.int32)


def kernel(x, rand_projs, codebook):
    b, n, dim = x.shape
    h, k, e = codebook.shape
    ntok = b * n
    xf = x.reshape(ntok, dim)
    p = rand_projs.reshape(dim, e)
    cb = codebook.reshape(k, e)
    grid = ntok // _BLK
    out = pl.pallas_call(
        _vq_kernel,
        grid=(grid,),
        in_specs=[
            pl.BlockSpec((_BLK, dim), lambda i: (i, 0)),
            pl.BlockSpec((dim, e), lambda i: (0, 0)),
            pl.BlockSpec((k, e), lambda i: (0, 0)),
        ],
        out_specs=pl.BlockSpec((1, 1, _BLK), lambda i: (i, 0, 0)),
        out_shape=jax.ShapeDtypeStruct((grid, 1, _BLK), jnp.int32),
        scratch_shapes=[pltpu.VMEM((k, e), jnp.float32)],
    )(xf, p, cb)
    return out.reshape(b, n)


# speedup vs baseline: 1.1782x; 1.0372x over previous
import jax
import jax.numpy as jnp
from jax.experimental import pallas as pl
from jax.experimental.pallas import tpu as pltpu

_BLK = 2304


def _vq_kernel(x_ref, p_ref, cb_ref, out_ref, cbn_ref):
    i = pl.program_id(0)

    @pl.when(i == 0)
    def _():
        cb = cb_ref[...]
        cn = jnp.sqrt(jnp.sum(cb * cb, axis=1, keepdims=True))
        cbn_ref[...] = cb / (cn + 1e-12)

    x = x_ref[...]
    p = p_ref[...]
    proj = jax.lax.dot_general(
        x, p, (((1,), (0,)), ((), ())), preferred_element_type=jnp.float32)
    qn = jnp.sqrt(jnp.sum(proj * proj, axis=1, keepdims=True))
    qx = proj / (qn + 1e-12)
    scores_t = jax.lax.dot_general(
        cbn_ref[...], qx, (((1,), (1,)), ((), ())),
        preferred_element_type=jnp.float32)
    idx = jnp.argmax(scores_t, axis=0)
    out_ref[...] = idx.reshape(1, 1, -1).astype(jnp.int32)


def kernel(x, rand_projs, codebook):
    b, n, dim = x.shape
    h, k, e = codebook.shape
    ntok = b * n
    xf = x.reshape(ntok, dim)
    p = rand_projs.reshape(dim, e)
    cb = codebook.reshape(k, e)
    grid = ntok // _BLK
    out = pl.pallas_call(
        _vq_kernel,
        grid=(grid,),
        in_specs=[
            pl.BlockSpec((_BLK, dim), lambda i: (i, 0)),
            pl.BlockSpec((dim, e), lambda i: (0, 0)),
            pl.BlockSpec((k, e), lambda i: (0, 0)),
        ],
        out_specs=pl.BlockSpec((1, 1, _BLK), lambda i: (i, 0, 0)),
        out_shape=jax.ShapeDtypeStruct((grid, 1, _BLK), jnp.int32),
        scratch_shapes=[pltpu.VMEM((k, e), jnp.float32)],
    )(xf, p, cb)
    return out.reshape(b, n)
